# baseline (speedup 1.0000x reference)
import functools

import jax
import jax.numpy as jnp
from jax import lax
from jax.experimental import pallas as pl
from jax.experimental.pallas import tpu as pltpu
from jax.experimental.pallas import tpu_sc as plsc

NUM_EXPERTS = 64
TOP_K = 8
BLOCK_R = 1024
N_WORKERS = 32
ROWS_PER_WORKER = 8192 // N_WORKERS


def _mm_kernel(x_ref, w_ref, o_ref):
    o_ref[...] = jax.lax.dot_general(
        x_ref[...], w_ref[...], (((1,), (1,)), ((), ())),
        preferred_element_type=jnp.float32,
    )


def _tc_logits(x, gate_weight):
    n, d = x.shape
    return pl.pallas_call(
        _mm_kernel,
        grid=(n // BLOCK_R,),
        in_specs=[
            pl.BlockSpec((BLOCK_R, d), lambda i: (i, 0)),
            pl.BlockSpec((NUM_EXPERTS, d), lambda i: (0, 0)),
        ],
        out_specs=pl.BlockSpec((BLOCK_R, NUM_EXPERTS), lambda i: (i, 0)),
        out_shape=jax.ShapeDtypeStruct((n, NUM_EXPERTS), jnp.float32),
    )(x, gate_weight)


def _merge(ka, va, kb, vb):
    kar = lax.rev(ka, (0,))
    var = lax.rev(va, (0,))
    sel = kar > kb
    hk = jnp.where(sel, kar, kb)
    hv = jnp.where(sel, var, vb)
    return plsc.sort_key_val(hk, hv, descending=True)


def _make_sc_topk(n_rows):
    rpw = n_rows // N_WORKERS

    def _sc_topk_body(lg_hbm, ow_hbm, oi_hbm, lg_v, w_v, i_v):
        wid = lax.axis_index("s") * 2 + lax.axis_index("c")
        base = wid * rpw
        pltpu.sync_copy(lg_hbm.at[pl.ds(base, rpw)], lg_v)
        iota = lax.iota(jnp.int32, 16)
        lane_lo = iota < TOP_K

        @plsc.parallel_loop(0, rpw, unroll=4)
        def row(r):
            s = [
                plsc.sort_key_val(
                    lg_v[r, pl.ds(16 * j, 16)], iota + 16 * j, descending=True
                )
                for j in range(4)
            ]
            m01 = _merge(*s[0], *s[1])
            m23 = _merge(*s[2], *s[3])
            kf, vf = _merge(*m01, *m23)
            mx = jnp.max(kf)
            e = jnp.exp(kf - mx)
            em = jnp.where(lane_lo, e, 0.0)
            w = em / jnp.sum(em)
            plsc.store_compressed(w_v.at[pl.ds(r * TOP_K, 16)], w, mask=lane_lo)
            plsc.store_compressed(i_v.at[pl.ds(r * TOP_K, 16)], vf, mask=lane_lo)
        out_elems = rpw * TOP_K
        pltpu.sync_copy(w_v.at[pl.ds(0, out_elems)],
                        ow_hbm.at[pl.ds(base * TOP_K, out_elems)])
        pltpu.sync_copy(i_v.at[pl.ds(0, out_elems)],
                        oi_hbm.at[pl.ds(base * TOP_K, out_elems)])

    return pl.kernel(
        _sc_topk_body,
        out_type=[
            jax.ShapeDtypeStruct((n_rows * TOP_K,), jnp.float32),
            jax.ShapeDtypeStruct((n_rows * TOP_K,), jnp.int32),
        ],
        mesh=plsc.VectorSubcoreMesh(
            core_axis_name="c", subcore_axis_name="s",
            num_cores=2, num_subcores=16,
        ),
        compiler_params=pltpu.CompilerParams(needs_layout_passes=False),
        scratch_types=[
            pltpu.VMEM((rpw, NUM_EXPERTS), jnp.float32),
            pltpu.VMEM((rpw * TOP_K + 16,), jnp.float32),
            pltpu.VMEM((rpw * TOP_K + 16,), jnp.int32),
        ],
    )


N_CHUNKS = 4


def kernel(x, gate_weight):
    n, _ = x.shape
    nc = n // N_CHUNKS
    sc_topk = _make_sc_topk(nc)
    ws, idxs = [], []
    for c in range(N_CHUNKS):
        logits = _tc_logits(x[c * nc:(c + 1) * nc], gate_weight)
        wf, idxf = sc_topk(logits)
        ws.append(wf.reshape(nc, TOP_K))
        idxs.append(idxf.reshape(nc, TOP_K))
    return jnp.concatenate(ws, axis=0), jnp.concatenate(idxs, axis=0)


# speedup vs baseline: 2.9030x; 2.9030x over previous
import jax
import jax.numpy as jnp
from jax.experimental import pallas as pl
from jax.experimental.pallas import tpu as pltpu

NUM_EXPERTS = 64
TOP_K = 8
BLOCK_R = 1024
D_SPLIT = 2


def _topk_write(l, ow_ref, oi_ref):
    iota = jax.lax.broadcasted_iota(jnp.int32, l.shape, 0)
    rev = (NUM_EXPERTS - 1) - iota
    vals = []
    idxs = []
    for _ in range(TOP_K):
        m = jnp.max(l, axis=0, keepdims=True)
        enc = jnp.where(l == m, rev, 0)
        idx = (NUM_EXPERTS - 1) - jnp.max(enc, axis=0, keepdims=True)
        vals.append(m)
        idxs.append(idx)
        l = jnp.where(iota == idx, -jnp.inf, l)
    v = jnp.concatenate(vals, axis=0)
    i = jnp.concatenate(idxs, axis=0)
    e = jnp.exp(v - v[0:1, :])
    w8 = e / jnp.sum(e, axis=0, keepdims=True)
    ow_ref[...] = w8.T
    oi_ref[...] = i.T


def _gate_kernel(x_ref, w_ref, ow_ref, oi_ref, acc_ref):
    di = pl.program_id(1)
    part = jax.lax.dot_general(
        w_ref[...], x_ref[...], (((1,), (1,)), ((), ())),
        preferred_element_type=jnp.float32,
    )

    @pl.when(di == 0)
    def _():
        acc_ref[...] = part

    @pl.when(di == D_SPLIT - 1)
    def _():
        _topk_write(acc_ref[...] + part, ow_ref, oi_ref)


def kernel(x, gate_weight):
    n, d = x.shape
    dc = d // D_SPLIT
    ow, oi = pl.pallas_call(
        _gate_kernel,
        grid=(n // BLOCK_R, D_SPLIT),
        in_specs=[
            pl.BlockSpec((BLOCK_R, dc), lambda i, j: (i, j)),
            pl.BlockSpec((NUM_EXPERTS, dc), lambda i, j: (0, j)),
        ],
        out_specs=[
            pl.BlockSpec((BLOCK_R, TOP_K), lambda i, j: (i, 0)),
            pl.BlockSpec((BLOCK_R, TOP_K), lambda i, j: (i, 0)),
        ],
        out_shape=[
            jax.ShapeDtypeStruct((n, TOP_K), jnp.float32),
            jax.ShapeDtypeStruct((n, TOP_K), jnp.int32),
        ],
        scratch_shapes=[pltpu.VMEM((NUM_EXPERTS, BLOCK_R), jnp.float32)],
    )(x, gate_weight)
    return ow, oi


# speedup vs baseline: 3.2878x; 1.1326x over previous
import jax
import jax.numpy as jnp
from jax.experimental import pallas as pl

NUM_EXPERTS = 64
TOP_K = 8
BLOCK_R = 1024


def _gate_kernel(x_ref, w_ref, ow_ref, oi_ref):
    logits = jax.lax.dot_general(
        w_ref[...], x_ref[...], (((1,), (1,)), ((), ())),
        preferred_element_type=jnp.float32,
    )
    iota = jax.lax.broadcasted_iota(jnp.int32, logits.shape, 0)
    rev = (NUM_EXPERTS - 1) - iota
    l = logits
    vals = []
    idxs = []
    for _ in range(TOP_K):
        m = jnp.max(l, axis=0, keepdims=True)
        enc = jnp.where(l == m, rev, 0)
        idx = (NUM_EXPERTS - 1) - jnp.max(enc, axis=0, keepdims=True)
        vals.append(m)
        idxs.append(idx)
        l = jnp.where(iota == idx, -jnp.inf, l)
    v = jnp.concatenate(vals, axis=0)
    i = jnp.concatenate(idxs, axis=0)
    e = jnp.exp(v - v[0:1, :])
    w8 = e / jnp.sum(e, axis=0, keepdims=True)
    ow_ref[...] = w8.T
    oi_ref[...] = i.T


def kernel(x, gate_weight):
    n, d = x.shape
    ow, oi = pl.pallas_call(
        _gate_kernel,
        grid=(n // BLOCK_R,),
        in_specs=[
            pl.BlockSpec((BLOCK_R, d), lambda i: (i, 0)),
            pl.BlockSpec((NUM_EXPERTS, d), lambda i: (0, 0)),
        ],
        out_specs=[
            pl.BlockSpec((BLOCK_R, TOP_K), lambda i: (i, 0)),
            pl.BlockSpec((BLOCK_R, TOP_K), lambda i: (i, 0)),
        ],
        out_shape=[
            jax.ShapeDtypeStruct((n, TOP_K), jnp.float32),
            jax.ShapeDtypeStruct((n, TOP_K), jnp.int32),
        ],
    )(x, gate_weight)
    return ow, oi
